# Initial kernel scaffold; baseline (speedup 1.0000x reference)
import jax
import jax.numpy as jnp
from jax.experimental import pallas as pl


def kernel(X):
    raise NotImplementedError("write your pallas kernel here")


# speedup vs baseline: 10.7254x; 10.7254x over previous
import functools

import jax
import jax.numpy as jnp
from jax.experimental import pallas as pl
from jax.experimental.pallas import tpu as pltpu

N = 8192
D = 512
BM = 128


def _knn_adj_kernel(xr_ref, xf_ref, x2r_ref, x2c_ref, out_ref):
    rows = xr_ref[...]
    g = jax.lax.dot_general(
        rows, xf_ref[...],
        dimension_numbers=(((1,), (1,)), ((), ())),
        preferred_element_type=jnp.float32,
    )
    dist = (x2r_ref[...] + x2c_ref[...]) - 2.0 * g
    jidx = jax.lax.broadcasted_iota(jnp.int32, (BM, N), 1)
    v1 = jnp.min(dist, axis=1, keepdims=True)
    i1 = jnp.min(jnp.where(dist == v1, jidx, N), axis=1, keepdims=True)
    m1 = jidx == i1
    v2 = jnp.min(jnp.where(m1, jnp.inf, dist), axis=1, keepdims=True)
    i2 = jnp.min(jnp.where((dist == v2) & ~m1, jidx, N), axis=1, keepdims=True)
    out_ref[...] = ((jidx == i1) | (jidx == i2)).astype(jnp.float32)


@functools.partial(jax.jit, static_argnames=("interpret",))
def kernel(X, interpret=False):
    x2 = jnp.sum(X * X, axis=1)
    x2_col = x2.reshape(N, 1)
    x2_row = x2.reshape(1, N)
    return pl.pallas_call(
        _knn_adj_kernel,
        grid=(N // BM,),
        in_specs=[
            pl.BlockSpec((BM, D), lambda i: (i, 0)),
            pl.BlockSpec((N, D), lambda i: (0, 0)),
            pl.BlockSpec((BM, 1), lambda i: (i, 0)),
            pl.BlockSpec((1, N), lambda i: (0, 0)),
        ],
        out_specs=pl.BlockSpec((BM, N), lambda i: (i, 0)),
        out_shape=jax.ShapeDtypeStruct((N, N), jnp.float32),
        interpret=interpret,
    )(X, X, x2_col, x2_row)


# speedup vs baseline: 12.6818x; 1.1824x over previous
import functools

import jax
import jax.numpy as jnp
from jax.experimental import pallas as pl
from jax.experimental.pallas import tpu as pltpu

N = 8192
D = 512
BM = 128


def _knn_adj_kernel(xr_ref, xf_ref, x2r_ref, x2c_ref, out_ref):
    rows = xr_ref[...]
    g = jax.lax.dot_general(
        rows, xf_ref[...],
        dimension_numbers=(((1,), (1,)), ((), ())),
        preferred_element_type=jnp.float32,
    )
    dist = (x2r_ref[...] + x2c_ref[...]) - 2.0 * g
    jidx = jax.lax.broadcasted_iota(jnp.int32, (BM, N), 1)
    i1 = jnp.argmin(dist, axis=1, keepdims=True).astype(jnp.int32)
    d2 = jnp.where(jidx == i1, jnp.inf, dist)
    i2 = jnp.argmin(d2, axis=1, keepdims=True).astype(jnp.int32)
    out_ref[...] = ((jidx == i1) | (jidx == i2)).astype(jnp.float32)


@functools.partial(jax.jit, static_argnames=("interpret",))
def kernel(X, interpret=False):
    x2 = jnp.sum(X * X, axis=1)
    x2_col = x2.reshape(N, 1)
    x2_row = x2.reshape(1, N)
    return pl.pallas_call(
        _knn_adj_kernel,
        grid=(N // BM,),
        in_specs=[
            pl.BlockSpec((BM, D), lambda i: (i, 0)),
            pl.BlockSpec((N, D), lambda i: (0, 0)),
            pl.BlockSpec((BM, 1), lambda i: (i, 0)),
            pl.BlockSpec((1, N), lambda i: (0, 0)),
        ],
        out_specs=pl.BlockSpec((BM, N), lambda i: (i, 0)),
        out_shape=jax.ShapeDtypeStruct((N, N), jnp.float32),
        interpret=interpret,
    )(X, X, x2_col, x2_row)


# speedup vs baseline: 19.5969x; 1.5453x over previous
import functools

import jax
import jax.numpy as jnp
from jax.experimental import pallas as pl
from jax.experimental.pallas import tpu as pltpu

N = 8192
D = 512
BM = 256


def _knn_adj_kernel(xr_ref, xf_ref, x2r_ref, x2c_ref, out_ref):
    rows = xr_ref[...]
    g = jax.lax.dot_general(
        rows, xf_ref[...],
        dimension_numbers=(((1,), (1,)), ((), ())),
        preferred_element_type=jnp.float32,
    )
    dist = (x2r_ref[...] + x2c_ref[...]) - 2.0 * g
    jidx = jax.lax.broadcasted_iota(jnp.int32, (BM, N), 1)
    i1 = jnp.argmin(dist, axis=1, keepdims=True).astype(jnp.int32)
    d2 = jnp.where(jidx == i1, jnp.inf, dist)
    i2 = jnp.argmin(d2, axis=1, keepdims=True).astype(jnp.int32)
    out_ref[...] = ((jidx == i1) | (jidx == i2)).astype(jnp.float32)


@functools.partial(jax.jit, static_argnames=("interpret",))
def kernel(X, interpret=False):
    x2 = jnp.sum(X * X, axis=1)
    x2_col = x2.reshape(N, 1)
    x2_row = x2.reshape(1, N)
    return pl.pallas_call(
        _knn_adj_kernel,
        grid=(N // BM,),
        in_specs=[
            pl.BlockSpec((BM, D), lambda i: (i, 0)),
            pl.BlockSpec((N, D), lambda i: (0, 0)),
            pl.BlockSpec((BM, 1), lambda i: (i, 0)),
            pl.BlockSpec((1, N), lambda i: (0, 0)),
        ],
        out_specs=pl.BlockSpec((BM, N), lambda i: (i, 0)),
        out_shape=jax.ShapeDtypeStruct((N, N), jnp.float32),
        interpret=interpret,
    )(X, X, x2_col, x2_row)
